# baseline (device time: 13673 ns/iter reference)
import os

import jax
import jax.numpy as jnp
from jax import lax
from jax.experimental import pallas as pl
from jax.experimental.pallas import tpu as pltpu

N_DEV = 16
CHUNKS = (1024, 1536, 1536)
N_CHUNK = len(CHUNKS)
_MODE = os.environ.get("KMODE", "full")


def kernel(x, W, labels):
    T, D = x.shape
    V_LOC = W.shape[1]
    assert sum(CHUNKS) == V_LOC
    offs = [sum(CHUNKS[:c]) for c in range(N_CHUNK)]
    x = pltpu.with_memory_space_constraint(x, pltpu.MemorySpace.HBM)
    W = pltpu.with_memory_space_constraint(W, pltpu.MemorySpace.HBM)
    labels = pltpu.with_memory_space_constraint(labels, pltpu.MemorySpace.HBM)

    def body(
        x_hbm,
        w_hbm,
        lab_hbm,
        out_hbm,
        x_ref,
        w_ref,
        lab_ref,
        out_ref,
        comm_ref,
        in_sems,
        w_sems,
        out_sem,
        send_sem,
        recv_sem,
    ):
        my = lax.axis_index("i")

        if _MODE == "comm":
            barrier_sem = pltpu.get_barrier_semaphore()
            for d in range(1, N_DEV):
                pl.semaphore_signal(
                    barrier_sem,
                    inc=1,
                    device_id=(lax.rem(my + d, N_DEV),),
                    device_id_type=pl.DeviceIdType.MESH,
                )
            comm_ref[0, :, :] = jnp.ones((3, T), jnp.bfloat16)
            pl.semaphore_wait(barrier_sem, N_DEV - 1)
            drain = None
            for d in range(N_DEV - 1, 0, -1):
                rdma = pltpu.make_async_remote_copy(
                    src_ref=comm_ref.at[0],
                    dst_ref=comm_ref.at[d],
                    send_sem=send_sem,
                    recv_sem=recv_sem,
                    device_id=(lax.rem(my + d, N_DEV),),
                    device_id_type=pl.DeviceIdType.MESH,
                )
                rdma.start()
                drain = rdma
            for _ in range(N_DEV - 1):
                drain.wait_recv()
            allm = comm_ref[:, 0, :].astype(jnp.float32)
            out_ref[:] = jnp.sum(allm, axis=0)
            out_cp = pltpu.make_async_copy(out_ref, out_hbm, out_sem)
            out_cp.start()
            for _ in range(N_DEV - 1):
                drain.wait_send()
            out_cp.wait()
            return

        x_cp = pltpu.make_async_copy(x_hbm, x_ref, in_sems.at[0])
        lab_cp = pltpu.make_async_copy(lab_hbm, lab_ref, in_sems.at[1])
        x_cp.start()
        lab_cp.start()
        w_cps = []
        for c in range(N_CHUNK):
            cols = pl.ds(offs[c], CHUNKS[c])
            cp = pltpu.make_async_copy(
                w_hbm.at[:, cols], w_ref.at[:, cols], w_sems.at[c]
            )
            cp.start()
            w_cps.append(cp)

        if _MODE != "compute":
            barrier_sem = pltpu.get_barrier_semaphore()
            for d in range(1, N_DEV):
                pl.semaphore_signal(
                    barrier_sem,
                    inc=1,
                    device_id=(lax.rem(my + d, N_DEV),),
                    device_id_type=pl.DeviceIdType.MESH,
                )

        x_cp.wait()
        xb = x_ref[:, :].astype(jnp.bfloat16)
        lab_cp.wait()
        local_tgt = lab_ref[:] - my * V_LOC

        ms, ss, labs = [], [], []
        for c in range(N_CHUNK):
            w_cps[c].wait()
            logits = jnp.dot(
                xb,
                w_ref[:, pl.ds(offs[c], CHUNKS[c])].astype(jnp.bfloat16),
                preferred_element_type=jnp.float32,
            ).astype(jnp.bfloat16)
            m_c = jnp.max(logits, axis=1)
            s_c = jnp.sum(
                jnp.exp(logits - m_c[:, None]), axis=1, dtype=jnp.float32
            )
            tgt_c = local_tgt - offs[c]
            col = lax.broadcasted_iota(jnp.int32, (T, CHUNKS[c]), 1)
            lab_c = jnp.sum(
                jnp.where(col == tgt_c[:, None], logits, jnp.bfloat16(0.0)),
                axis=1,
                dtype=jnp.float32,
            )
            ms.append(m_c.astype(jnp.float32))
            ss.append(s_c)
            labs.append(lab_c)

        cm = jnp.stack(ms)
        m = jnp.max(cm, axis=0)
        s = jnp.sum(jnp.stack(ss) * jnp.exp(cm - m[None, :]), axis=0)
        lab = jnp.sum(jnp.stack(labs), axis=0)

        comm_ref[0, 0, :] = m.astype(jnp.bfloat16)
        comm_ref[0, 1, :] = s.astype(jnp.bfloat16)
        comm_ref[0, 2, :] = lab.astype(jnp.bfloat16)

        if _MODE != "compute":
            pl.semaphore_wait(barrier_sem, N_DEV - 1)

            drain = None
            for d in range(N_DEV - 1, 0, -1):
                rdma = pltpu.make_async_remote_copy(
                    src_ref=comm_ref.at[0],
                    dst_ref=comm_ref.at[d],
                    send_sem=send_sem,
                    recv_sem=recv_sem,
                    device_id=(lax.rem(my + d, N_DEV),),
                    device_id_type=pl.DeviceIdType.MESH,
                )
                rdma.start()
                drain = rdma
            for _ in range(N_DEV - 1):
                drain.wait_recv()

        allm = comm_ref[:, 0, :].astype(jnp.float32)
        alls = comm_ref[:, 1, :].astype(jnp.float32)
        alllab = comm_ref[:, 2, :].astype(jnp.float32)
        M = jnp.max(allm, axis=0)
        Z = jnp.sum(alls * jnp.exp(allm - M[None, :]), axis=0)
        lab_tot = jnp.sum(alllab, axis=0)
        out_ref[:] = M + jnp.log(Z) - lab_tot

        out_cp = pltpu.make_async_copy(out_ref, out_hbm, out_sem)
        out_cp.start()
        if _MODE != "compute":
            for _ in range(N_DEV - 1):
                drain.wait_send()
        out_cp.wait()

    return pl.pallas_call(
        body,
        out_shape=jax.ShapeDtypeStruct((T,), jnp.float32),
        in_specs=[
            pl.BlockSpec(memory_space=pl.ANY),
            pl.BlockSpec(memory_space=pl.ANY),
            pl.BlockSpec(memory_space=pl.ANY),
        ],
        out_specs=pl.BlockSpec(memory_space=pl.ANY),
        scratch_shapes=[
            pltpu.VMEM((T, D), jnp.float32),
            pltpu.VMEM((D, V_LOC), jnp.float32),
            pltpu.VMEM((T,), jnp.int32),
            pltpu.VMEM((T,), jnp.float32),
            pltpu.VMEM((N_DEV, 3, T), jnp.bfloat16),
            pltpu.SemaphoreType.DMA((2,)),
            pltpu.SemaphoreType.DMA((N_CHUNK,)),
            pltpu.SemaphoreType.DMA(()),
            pltpu.SemaphoreType.DMA(()),
            pltpu.SemaphoreType.DMA(()),
        ],
        **(
            {}
            if _MODE == "compute"
            else dict(compiler_params=pltpu.CompilerParams(collective_id=0))
        ),
    )(x, W, labels)


# device time: 13665 ns/iter; 1.0006x vs baseline; 1.0006x over previous
import os

import jax
import jax.numpy as jnp
from jax import lax
from jax.experimental import pallas as pl
from jax.experimental.pallas import tpu as pltpu

N_DEV = 16
CHUNKS = (1024, 1536, 1536)
N_CHUNK = len(CHUNKS)
_MODE = os.environ.get("KMODE", "full")


def kernel(x, W, labels):
    T, D = x.shape
    V_LOC = W.shape[1]
    assert sum(CHUNKS) == V_LOC
    offs = [sum(CHUNKS[:c]) for c in range(N_CHUNK)]
    x = pltpu.with_memory_space_constraint(x, pltpu.MemorySpace.HBM)
    W = pltpu.with_memory_space_constraint(W, pltpu.MemorySpace.HBM)
    labels = pltpu.with_memory_space_constraint(labels, pltpu.MemorySpace.HBM)

    def body(
        x_hbm,
        w_hbm,
        lab_hbm,
        out_hbm,
        x_ref,
        w_ref,
        lab_ref,
        out_ref,
        comm_ref,
        in_sems,
        w_sems,
        out_sem,
        send_sem,
        recv_sem,
        p2_send_sem,
        p2_recv_sem,
    ):
        my = lax.axis_index("i")

        if _MODE == "comm":
            barrier_sem = pltpu.get_barrier_semaphore()
            for d in range(1, N_DEV):
                pl.semaphore_signal(
                    barrier_sem,
                    inc=1,
                    device_id=(lax.rem(my + d, N_DEV),),
                    device_id_type=pl.DeviceIdType.MESH,
                )
            comm_ref[0, :, :] = jnp.ones((3, T), jnp.bfloat16)
            pl.semaphore_wait(barrier_sem, N_DEV - 1)
            drain = None
            for d in range(N_DEV - 1, 0, -1):
                rdma = pltpu.make_async_remote_copy(
                    src_ref=comm_ref.at[0],
                    dst_ref=comm_ref.at[d],
                    send_sem=send_sem,
                    recv_sem=recv_sem,
                    device_id=(lax.rem(my + d, N_DEV),),
                    device_id_type=pl.DeviceIdType.MESH,
                )
                rdma.start()
                drain = rdma
            for _ in range(N_DEV - 1):
                drain.wait_recv()
            allm = comm_ref[:, 0, :].astype(jnp.float32)
            out_ref[:] = jnp.sum(allm, axis=0)
            out_cp = pltpu.make_async_copy(out_ref, out_hbm, out_sem)
            out_cp.start()
            for _ in range(N_DEV - 1):
                drain.wait_send()
            out_cp.wait()
            return

        x_cp = pltpu.make_async_copy(x_hbm, x_ref, in_sems.at[0])
        lab_cp = pltpu.make_async_copy(lab_hbm, lab_ref, in_sems.at[1])
        x_cp.start()
        lab_cp.start()
        w_cps = []
        for c in range(N_CHUNK):
            cols = pl.ds(offs[c], CHUNKS[c])
            cp = pltpu.make_async_copy(
                w_hbm.at[:, cols], w_ref.at[:, cols], w_sems.at[c]
            )
            cp.start()
            w_cps.append(cp)

        if _MODE != "compute":
            barrier_sem = pltpu.get_barrier_semaphore()
            for d in range(1, N_DEV):
                pl.semaphore_signal(
                    barrier_sem,
                    inc=1,
                    device_id=(lax.rem(my + d, N_DEV),),
                    device_id_type=pl.DeviceIdType.MESH,
                )

        x_cp.wait()
        xb = x_ref[:, :].astype(jnp.bfloat16)
        lab_cp.wait()
        local_tgt = lab_ref[:] - my * V_LOC

        ms, ss, labs = [], [], []
        for c in range(N_CHUNK):
            w_cps[c].wait()
            logits = jnp.dot(
                xb,
                w_ref[:, pl.ds(offs[c], CHUNKS[c])].astype(jnp.bfloat16),
                preferred_element_type=jnp.float32,
            ).astype(jnp.bfloat16)
            m_c = jnp.max(logits, axis=1)
            s_c = jnp.sum(
                jnp.exp(logits - m_c[:, None]), axis=1, dtype=jnp.float32
            )
            tgt_c = local_tgt - offs[c]
            col = lax.broadcasted_iota(jnp.int32, (T, CHUNKS[c]), 1)
            lab_c = jnp.sum(
                jnp.where(col == tgt_c[:, None], logits, jnp.bfloat16(0.0)),
                axis=1,
                dtype=jnp.float32,
            )
            ms.append(m_c.astype(jnp.float32))
            ss.append(s_c)
            labs.append(lab_c)

        cm = jnp.stack(ms)
        m = jnp.max(cm, axis=0)
        s = jnp.sum(jnp.stack(ss) * jnp.exp(cm - m[None, :]), axis=0)
        lab = jnp.sum(jnp.stack(labs), axis=0)

        comm_ref[0, 0, :] = m.astype(jnp.bfloat16)
        comm_ref[0, 1, :] = s.astype(jnp.bfloat16)
        comm_ref[0, 2, :] = lab.astype(jnp.bfloat16)

        def _combine(lo, hi):
            gm = comm_ref[lo:hi, 0, :].astype(jnp.float32)
            gs = comm_ref[lo:hi, 1, :].astype(jnp.float32)
            gl = comm_ref[lo:hi, 2, :].astype(jnp.float32)
            Mx = jnp.max(gm, axis=0)
            Zx = jnp.sum(gs * jnp.exp(gm - Mx[None, :]), axis=0)
            Lx = jnp.sum(gl, axis=0)
            return Mx, Zx, Lx

        p1_drain = p2_drain = None
        if _MODE == "hier":
            pl.semaphore_wait(barrier_sem, N_DEV - 1)
            r = lax.rem(my, 4)
            base = my - r
            for j in range(3, 0, -1):
                rdma = pltpu.make_async_remote_copy(
                    src_ref=comm_ref.at[0],
                    dst_ref=comm_ref.at[j],
                    send_sem=send_sem,
                    recv_sem=recv_sem,
                    device_id=(base + lax.rem(r + j, 4),),
                    device_id_type=pl.DeviceIdType.MESH,
                )
                rdma.start()
                p1_drain = rdma
            for _ in range(3):
                p1_drain.wait_recv()
            Mp, Zp, Lp = _combine(0, 4)
            comm_ref[4, 0, :] = Mp.astype(jnp.bfloat16)
            comm_ref[4, 1, :] = Zp.astype(jnp.bfloat16)
            comm_ref[4, 2, :] = Lp.astype(jnp.bfloat16)
            for k in range(3, 0, -1):
                rdma = pltpu.make_async_remote_copy(
                    src_ref=comm_ref.at[4],
                    dst_ref=comm_ref.at[4 + k],
                    send_sem=p2_send_sem,
                    recv_sem=p2_recv_sem,
                    device_id=(lax.rem(my + 4 * k, N_DEV),),
                    device_id_type=pl.DeviceIdType.MESH,
                )
                rdma.start()
                p2_drain = rdma
            for _ in range(3):
                p2_drain.wait_recv()
            M, Z, lab_tot = _combine(4, 8)
        elif _MODE != "compute":
            pl.semaphore_wait(barrier_sem, N_DEV - 1)

            for d in range(N_DEV - 1, 0, -1):
                rdma = pltpu.make_async_remote_copy(
                    src_ref=comm_ref.at[0],
                    dst_ref=comm_ref.at[d],
                    send_sem=send_sem,
                    recv_sem=recv_sem,
                    device_id=(lax.rem(my + d, N_DEV),),
                    device_id_type=pl.DeviceIdType.MESH,
                )
                rdma.start()
                p1_drain = rdma
            for _ in range(N_DEV - 1):
                p1_drain.wait_recv()
            M, Z, lab_tot = _combine(0, N_DEV)
        else:
            M, Z, lab_tot = _combine(0, 1)

        out_ref[:] = M + jnp.log(Z) - lab_tot

        out_cp = pltpu.make_async_copy(out_ref, out_hbm, out_sem)
        out_cp.start()
        if _MODE == "hier":
            for _ in range(3):
                p1_drain.wait_send()
                p2_drain.wait_send()
        elif _MODE != "compute":
            for _ in range(N_DEV - 1):
                p1_drain.wait_send()
        out_cp.wait()

    return pl.pallas_call(
        body,
        out_shape=jax.ShapeDtypeStruct((T,), jnp.float32),
        in_specs=[
            pl.BlockSpec(memory_space=pl.ANY),
            pl.BlockSpec(memory_space=pl.ANY),
            pl.BlockSpec(memory_space=pl.ANY),
        ],
        out_specs=pl.BlockSpec(memory_space=pl.ANY),
        scratch_shapes=[
            pltpu.VMEM((T, D), jnp.float32),
            pltpu.VMEM((D, V_LOC), jnp.float32),
            pltpu.VMEM((T,), jnp.int32),
            pltpu.VMEM((T,), jnp.float32),
            pltpu.VMEM((N_DEV, 3, T), jnp.bfloat16),
            pltpu.SemaphoreType.DMA((2,)),
            pltpu.SemaphoreType.DMA((N_CHUNK,)),
            pltpu.SemaphoreType.DMA(()),
            pltpu.SemaphoreType.DMA(()),
            pltpu.SemaphoreType.DMA(()),
            pltpu.SemaphoreType.DMA(()),
            pltpu.SemaphoreType.DMA(()),
        ],
        **(
            {}
            if _MODE == "compute"
            else dict(compiler_params=pltpu.CompilerParams(collective_id=0))
        ),
    )(x, W, labels)
